# baseline (speedup 1.0000x reference)
import jax
import jax.numpy as jnp
from jax.experimental import pallas as pl
from jax.experimental.pallas import tpu as pltpu

_ACTION_DIM = 24
_TM = 8192


def _round_up(n, m):
    return ((n + m - 1) // m) * m


def _mlp_kernel(x_ref, w1_ref, b1_ref, w2_ref, b2_ref, w3_ref, b3_ref, o_ref,
                w1s, w2s, w3s):
    h2p = w2_ref.shape[1]
    outp = w3_ref.shape[1]

    @pl.when(pl.program_id(0) == 0)
    def _cast_weights():
        w1s[...] = w1_ref[...].astype(jnp.bfloat16)
        w2s[...] = w2_ref[...].astype(jnp.bfloat16)
        w3b = w3_ref[...].astype(jnp.bfloat16)
        zero = jnp.zeros((h2p, outp), jnp.bfloat16)
        w3s[...] = jnp.concatenate(
            [jnp.concatenate([w3b, zero], axis=1),
             jnp.concatenate([zero, w3b], axis=1)], axis=0)

    x = x_ref[...].astype(jnp.bfloat16)

    h1 = jnp.dot(x, w1s[...], preferred_element_type=jnp.float32) + b1_ref[...]
    h1 = jnp.maximum(h1, 0.0).astype(jnp.bfloat16)

    h2 = jnp.dot(h1, w2s[...], preferred_element_type=jnp.float32) + b2_ref[...]
    h2 = jnp.maximum(h2, 0.0).astype(jnp.bfloat16)

    half = h2.shape[0] // 2
    hh = jnp.concatenate([h2[:half], h2[half:]], axis=1)
    res = jnp.dot(hh, w3s[...], preferred_element_type=jnp.float32)

    na = o_ref.shape[1]
    b3 = b3_ref[:, :na]
    o_ref[:half, :] = jnp.tanh(res[:, :na] + b3).astype(o_ref.dtype)
    o_ref[half:, :] = jnp.tanh(res[:, outp : outp + na] + b3).astype(o_ref.dtype)


def kernel(x, w1p, b1p, w2p, b2p, w3p, b3p):
    B, sp = x.shape
    h1p = w1p.shape[1]
    h2p = w2p.shape[1]
    outp = w3p.shape[1]

    tm = min(_TM, _round_up(B, 8))
    B_pad = _round_up(B, tm)
    xp = jnp.pad(x, ((0, B_pad - B), (0, 0))) if B_pad != B else x

    out = pl.pallas_call(
        _mlp_kernel,
        out_shape=jax.ShapeDtypeStruct((B_pad, _ACTION_DIM), jnp.bfloat16),
        grid=(B_pad // tm,),
        in_specs=[
            pl.BlockSpec((tm, sp), lambda i: (i, 0)),
            pl.BlockSpec((sp, h1p), lambda i: (0, 0)),
            pl.BlockSpec((1, h1p), lambda i: (0, 0)),
            pl.BlockSpec((h1p, h2p), lambda i: (0, 0)),
            pl.BlockSpec((1, h2p), lambda i: (0, 0)),
            pl.BlockSpec((h2p, outp), lambda i: (0, 0)),
            pl.BlockSpec((1, outp), lambda i: (0, 0)),
        ],
        out_specs=pl.BlockSpec((tm, _ACTION_DIM), lambda i: (i, 0)),
        scratch_shapes=[
            pltpu.VMEM((sp, h1p), jnp.bfloat16),
            pltpu.VMEM((h1p, h2p), jnp.bfloat16),
            pltpu.VMEM((2 * h2p, 2 * outp), jnp.bfloat16),
        ],
        compiler_params=pltpu.CompilerParams(
            dimension_semantics=("arbitrary",),
        ),
    )(xp, w1p, b1p, w2p, b2p, w3p, b3p)

    out = out[:B] if B_pad != B else out
    return out.astype(jnp.float32)


# speedup vs baseline: 1.0022x; 1.0022x over previous
import jax
import jax.numpy as jnp
from jax.experimental import pallas as pl
from jax.experimental.pallas import tpu as pltpu

_ACTION_DIM = 24
_TM = 4096


def _round_up(n, m):
    return ((n + m - 1) // m) * m


def _mlp_kernel(x_ref, w1_ref, b1_ref, w2_ref, b2_ref, w3_ref, b3_ref, o_ref,
                w1s, w2s, w3s):
    @pl.when(pl.program_id(0) == 0)
    def _cast_weights():
        w1s[...] = w1_ref[...].astype(jnp.bfloat16)
        w2s[...] = w2_ref[...].astype(jnp.bfloat16)
        w3s[...] = w3_ref[...].astype(jnp.bfloat16)

    x = x_ref[...].astype(jnp.bfloat16)

    h1 = jnp.dot(x, w1s[...], preferred_element_type=jnp.float32) + b1_ref[...]
    h1 = jnp.maximum(h1, 0.0).astype(jnp.bfloat16)

    h2 = jnp.dot(h1, w2s[...], preferred_element_type=jnp.float32) + b2_ref[...]
    h2 = jnp.maximum(h2, 0.0).astype(jnp.bfloat16)

    out = jnp.dot(h2, w3s[...], preferred_element_type=jnp.float32) + b3_ref[...]
    o_ref[...] = jnp.tanh(out[:, : o_ref.shape[1]]).astype(o_ref.dtype)


def kernel(x, w1p, b1p, w2p, b2p, w3p, b3p):
    B, sp = x.shape
    h1p = w1p.shape[1]
    h2p = w2p.shape[1]
    outp = w3p.shape[1]

    tm = min(_TM, _round_up(B, 8))
    B_pad = _round_up(B, tm)
    xp = jnp.pad(x, ((0, B_pad - B), (0, 0))) if B_pad != B else x

    out = pl.pallas_call(
        _mlp_kernel,
        out_shape=jax.ShapeDtypeStruct((B_pad, _ACTION_DIM), jnp.bfloat16),
        grid=(B_pad // tm,),
        in_specs=[
            pl.BlockSpec((tm, sp), lambda i: (i, 0)),
            pl.BlockSpec((sp, h1p), lambda i: (0, 0)),
            pl.BlockSpec((1, h1p), lambda i: (0, 0)),
            pl.BlockSpec((h1p, h2p), lambda i: (0, 0)),
            pl.BlockSpec((1, h2p), lambda i: (0, 0)),
            pl.BlockSpec((h2p, outp), lambda i: (0, 0)),
            pl.BlockSpec((1, outp), lambda i: (0, 0)),
        ],
        out_specs=pl.BlockSpec((tm, _ACTION_DIM), lambda i: (i, 0)),
        scratch_shapes=[
            pltpu.VMEM((sp, h1p), jnp.bfloat16),
            pltpu.VMEM((h1p, h2p), jnp.bfloat16),
            pltpu.VMEM((h2p, outp), jnp.bfloat16),
        ],
        compiler_params=pltpu.CompilerParams(
            dimension_semantics=("arbitrary",),
        ),
    )(xp, w1p, b1p, w2p, b2p, w3p, b3p)

    out = out[:B] if B_pad != B else out
    return out.astype(jnp.float32)


# speedup vs baseline: 1.0147x; 1.0125x over previous
import jax
import jax.numpy as jnp
from jax.experimental import pallas as pl
from jax.experimental.pallas import tpu as pltpu

_ACTION_DIM = 24
_TM = 8192


def _round_up(n, m):
    return ((n + m - 1) // m) * m


def _mlp_kernel(x_ref, w1_ref, b1_ref, w2_ref, b2_ref, w3_ref, b3_ref, o_ref,
                w1s, w2s, w3s):
    @pl.when(pl.program_id(0) == 0)
    def _cast_weights():
        w1s[...] = w1_ref[...].astype(jnp.bfloat16)
        w2s[...] = w2_ref[...].astype(jnp.bfloat16)
        w3s[...] = w3_ref[...].astype(jnp.bfloat16)

    x = x_ref[...].astype(jnp.bfloat16)

    h1 = jnp.dot(x, w1s[...], preferred_element_type=jnp.float32) + b1_ref[...]
    h1 = jnp.maximum(h1, 0.0).astype(jnp.bfloat16)

    h2 = jnp.dot(h1, w2s[...], preferred_element_type=jnp.float32) + b2_ref[...]
    h2 = jnp.maximum(h2, 0.0).astype(jnp.bfloat16)

    out = jnp.dot(h2, w3s[...], preferred_element_type=jnp.float32) + b3_ref[...]
    o_ref[...] = jnp.tanh(out[:, : o_ref.shape[1]]).astype(o_ref.dtype)


def kernel(x, w1p, b1p, w2p, b2p, w3p, b3p):
    B, sp = x.shape
    h1p = w1p.shape[1]
    h2p = w2p.shape[1]
    outp = w3p.shape[1]

    tm = min(_TM, _round_up(B, 8))
    B_pad = _round_up(B, tm)
    xp = jnp.pad(x, ((0, B_pad - B), (0, 0))) if B_pad != B else x

    out = pl.pallas_call(
        _mlp_kernel,
        out_shape=jax.ShapeDtypeStruct((B_pad, _ACTION_DIM), jnp.bfloat16),
        grid=(B_pad // tm,),
        in_specs=[
            pl.BlockSpec((tm, sp), lambda i: (i, 0)),
            pl.BlockSpec((sp, h1p), lambda i: (0, 0)),
            pl.BlockSpec((1, h1p), lambda i: (0, 0)),
            pl.BlockSpec((h1p, h2p), lambda i: (0, 0)),
            pl.BlockSpec((1, h2p), lambda i: (0, 0)),
            pl.BlockSpec((h2p, outp), lambda i: (0, 0)),
            pl.BlockSpec((1, outp), lambda i: (0, 0)),
        ],
        out_specs=pl.BlockSpec((tm, _ACTION_DIM), lambda i: (i, 0)),
        scratch_shapes=[
            pltpu.VMEM((sp, h1p), jnp.bfloat16),
            pltpu.VMEM((h1p, h2p), jnp.bfloat16),
            pltpu.VMEM((h2p, outp), jnp.bfloat16),
        ],
        compiler_params=pltpu.CompilerParams(
            dimension_semantics=("arbitrary",),
        ),
    )(xp, w1p, b1p, w2p, b2p, w3p, b3p)

    out = out[:B] if B_pad != B else out
    return out.astype(jnp.float32)
